# baseline (speedup 1.0000x reference)
import functools

import jax
import jax.numpy as jnp
from jax.experimental import pallas as pl
from jax.experimental.pallas import tpu as pltpu

_N = 4096
_H = 128
_BM = 512


def _fused_kernel(pf_ref, t_ref, wp_ref, bp_ref, wt_ref, bt_ref, wo_ref,
                  bo_ref, e_ref, out_ref, v_ref):
    m = pl.program_id(0)

    @pl.when(m == 0)
    def _compute_v():
        ph = jnp.dot(pf_ref[...], wp_ref[...].T,
                     preferred_element_type=jnp.float32,
                     precision=jax.lax.Precision.HIGHEST)
        th = t_ref[...] * wt_ref[...]
        h = ph + bp_ref[...] + th + bt_ref[...]
        h = jnp.clip(h, -1000000.0, 1000000.0)
        v_ref[...] = jnp.dot(h, wo_ref[...].T,
                             preferred_element_type=jnp.float32,
                             precision=jax.lax.Precision.HIGHEST)

    out_ref[...] = (
        jnp.dot(e_ref[...], v_ref[...], preferred_element_type=jnp.float32,
                precision=jax.lax.Precision.HIGHEST)
        + bo_ref[...]
    )


def kernel(policy_features, traffic_features, edge_index, W_policy, b_policy,
           W_traffic, b_traffic, W_out, b_out):
    t_col = traffic_features.reshape(_N, 1)
    wt_row = W_traffic.reshape(1, _H)
    bp_row = b_policy.reshape(1, _H)
    bt_row = b_traffic.reshape(1, _H)
    bo_11 = b_out.reshape(1, 1)

    n_blocks = _N // _BM
    const_spec = lambda shape: pl.BlockSpec(shape, lambda m: (0, 0))

    return pl.pallas_call(
        _fused_kernel,
        grid=(n_blocks,),
        in_specs=[
            const_spec((_N, _H)),
            const_spec((_N, 1)),
            const_spec((_H, _H)),
            const_spec((1, _H)),
            const_spec((1, _H)),
            const_spec((1, _H)),
            const_spec((1, _H)),
            const_spec((1, 1)),
            pl.BlockSpec((_BM, _N), lambda m: (m, 0)),
        ],
        out_specs=pl.BlockSpec((_BM, 1), lambda m: (m, 0)),
        out_shape=jax.ShapeDtypeStruct((_N, 1), jnp.float32),
        scratch_shapes=[pltpu.VMEM((_N, 1), jnp.float32)],
    )(policy_features, t_col, W_policy, bp_row, wt_row, bt_row, W_out, bo_11,
      edge_index)


# speedup vs baseline: 2.1847x; 2.1847x over previous
import jax
import jax.numpy as jnp
from jax.experimental import pallas as pl
from jax.experimental.pallas import tpu as pltpu

_N = 4096
_H = 128
_BM = 512
_CHUNKS = _N // _H


def _fused_kernel(pf_ref, t_ref, wp_ref, bp_ref, wt_ref, bt_ref, wo_ref,
                  bo_ref, e_ref, out_ref, v_ref):
    m = pl.program_id(0)

    @pl.when(m == 0)
    def _compute_v():
        ph = jnp.dot(pf_ref[...], wp_ref[...].T,
                     preferred_element_type=jnp.float32,
                     precision=jax.lax.Precision.HIGHEST)
        th = t_ref[...] * wt_ref[...]
        h = ph + bp_ref[...] + th + bt_ref[...]
        h = jnp.clip(h, -1000000.0, 1000000.0)
        h3 = h.reshape(_CHUNKS, _H, _H)
        v_ref[...] = jnp.sum(h3 * wo_ref[...].reshape(1, 1, _H), axis=2)

    e = e_ref[...]
    acc = e[:, 0:_H] * v_ref[0:1, :]
    for k in range(1, _CHUNKS):
        acc = acc + e[:, k * _H:(k + 1) * _H] * v_ref[k:k + 1, :]
    out_ref[...] = jnp.sum(acc, axis=1, keepdims=True) + bo_ref[...]


def kernel(policy_features, traffic_features, edge_index, W_policy, b_policy,
           W_traffic, b_traffic, W_out, b_out):
    t_col = traffic_features.reshape(_N, 1)
    wt_row = W_traffic.reshape(1, _H)
    bp_row = b_policy.reshape(1, _H)
    bt_row = b_traffic.reshape(1, _H)
    bo_11 = b_out.reshape(1, 1)

    n_blocks = _N // _BM
    const_spec = lambda shape: pl.BlockSpec(shape, lambda m: (0, 0))

    return pl.pallas_call(
        _fused_kernel,
        grid=(n_blocks,),
        in_specs=[
            const_spec((_N, _H)),
            const_spec((_N, 1)),
            const_spec((_H, _H)),
            const_spec((1, _H)),
            const_spec((1, _H)),
            const_spec((1, _H)),
            const_spec((1, _H)),
            const_spec((1, 1)),
            pl.BlockSpec((_BM, _N), lambda m: (m, 0)),
        ],
        out_specs=pl.BlockSpec((_BM, 1), lambda m: (m, 0)),
        out_shape=jax.ShapeDtypeStruct((_N, 1), jnp.float32),
        scratch_shapes=[pltpu.VMEM((_CHUNKS, _H), jnp.float32)],
    )(policy_features, t_col, W_policy, bp_row, wt_row, bt_row, W_out, bo_11,
      edge_index)


# speedup vs baseline: 2.1977x; 1.0060x over previous
import jax
import jax.numpy as jnp
from jax.experimental import pallas as pl
from jax.experimental.pallas import tpu as pltpu

_N = 4096
_H = 128
_BM = 512
_CHUNKS = _N // _H


def _fused_kernel(pf_ref, t_ref, wp_ref, bp_ref, wt_ref, bt_ref, wo_ref,
                  bo_ref, e_ref, out_ref, v_ref):
    m = pl.program_id(0)

    @pl.when(m == 0)
    def _compute_v():
        pf_b = pf_ref[...].astype(jnp.bfloat16)
        wp_b = wp_ref[...].astype(jnp.bfloat16)
        ph = jnp.dot(pf_b, wp_b.T, preferred_element_type=jnp.float32)
        th = t_ref[...] * wt_ref[...]
        h = ph + bp_ref[...] + th + bt_ref[...]
        h = jnp.clip(h, -1000000.0, 1000000.0)
        h3 = h.astype(jnp.bfloat16).astype(jnp.float32).reshape(_CHUNKS, _H, _H)
        wo_b = wo_ref[...].astype(jnp.bfloat16).astype(jnp.float32)
        v_ref[...] = jnp.sum(h3 * wo_b.reshape(1, 1, _H), axis=2)

    e = e_ref[...].astype(jnp.bfloat16).astype(jnp.float32)
    acc = e[:, 0:_H] * v_ref[0:1, :]
    for k in range(1, _CHUNKS):
        acc = acc + e[:, k * _H:(k + 1) * _H] * v_ref[k:k + 1, :]
    out_ref[...] = jnp.sum(acc, axis=1, keepdims=True) + bo_ref[...]


def kernel(policy_features, traffic_features, edge_index, W_policy, b_policy,
           W_traffic, b_traffic, W_out, b_out):
    t_col = traffic_features.reshape(_N, 1)
    wt_row = W_traffic.reshape(1, _H)
    bp_row = b_policy.reshape(1, _H)
    bt_row = b_traffic.reshape(1, _H)
    bo_11 = b_out.reshape(1, 1)

    n_blocks = _N // _BM
    const_spec = lambda shape: pl.BlockSpec(shape, lambda m: (0, 0))

    return pl.pallas_call(
        _fused_kernel,
        grid=(n_blocks,),
        in_specs=[
            const_spec((_N, _H)),
            const_spec((_N, 1)),
            const_spec((_H, _H)),
            const_spec((1, _H)),
            const_spec((1, _H)),
            const_spec((1, _H)),
            const_spec((1, _H)),
            const_spec((1, 1)),
            pl.BlockSpec((_BM, _N), lambda m: (m, 0)),
        ],
        out_specs=pl.BlockSpec((_BM, 1), lambda m: (m, 0)),
        out_shape=jax.ShapeDtypeStruct((_N, 1), jnp.float32),
        scratch_shapes=[pltpu.VMEM((_CHUNKS, _H), jnp.float32)],
    )(policy_features, t_col, W_policy, bp_row, wt_row, bt_row, W_out, bo_11,
      edge_index)


# speedup vs baseline: 2.2690x; 1.0324x over previous
import jax
import jax.numpy as jnp
from jax.experimental import pallas as pl
from jax.experimental.pallas import tpu as pltpu

_N = 4096
_H = 128
_BM = 512
_CHUNKS = _N // _H


def _fused_kernel(pf_ref, t_ref, wp_ref, bp_ref, wt_ref, bt_ref, wo_ref,
                  bo_ref, e_ref, out_ref, v_ref):
    m = pl.program_id(0)

    @pl.when(m == 0)
    def _compute_v():
        pf_b = pf_ref[...].astype(jnp.bfloat16)
        wp_b = wp_ref[...].astype(jnp.bfloat16)
        ph = jnp.dot(pf_b, wp_b.T, preferred_element_type=jnp.float32)
        th = t_ref[...] * wt_ref[...]
        h = ph + bp_ref[...] + th + bt_ref[...]
        h = jnp.clip(h, -1000000.0, 1000000.0)
        h3 = h.astype(jnp.bfloat16).astype(jnp.float32).reshape(_CHUNKS, _H, _H)
        wo_b = wo_ref[...].astype(jnp.bfloat16).astype(jnp.float32)
        v_ref[...] = jnp.sum(h3 * wo_b.reshape(1, 1, _H), axis=2)

    e = e_ref[...]
    acc = e[:, 0:_H] * v_ref[0:1, :]
    for k in range(1, _CHUNKS):
        acc = acc + e[:, k * _H:(k + 1) * _H] * v_ref[k:k + 1, :]
    out_ref[...] = jnp.sum(acc, axis=1, keepdims=True) + bo_ref[...]


def kernel(policy_features, traffic_features, edge_index, W_policy, b_policy,
           W_traffic, b_traffic, W_out, b_out):
    t_col = traffic_features.reshape(_N, 1)
    wt_row = W_traffic.reshape(1, _H)
    bp_row = b_policy.reshape(1, _H)
    bt_row = b_traffic.reshape(1, _H)
    bo_11 = b_out.reshape(1, 1)

    n_blocks = _N // _BM
    const_spec = lambda shape: pl.BlockSpec(shape, lambda m: (0, 0))

    return pl.pallas_call(
        _fused_kernel,
        grid=(n_blocks,),
        in_specs=[
            const_spec((_N, _H)),
            const_spec((_N, 1)),
            const_spec((_H, _H)),
            const_spec((1, _H)),
            const_spec((1, _H)),
            const_spec((1, _H)),
            const_spec((1, _H)),
            const_spec((1, 1)),
            pl.BlockSpec((_BM, _N), lambda m: (m, 0)),
        ],
        out_specs=pl.BlockSpec((_BM, 1), lambda m: (m, 0)),
        out_shape=jax.ShapeDtypeStruct((_N, 1), jnp.float32),
        scratch_shapes=[pltpu.VMEM((_CHUNKS, _H), jnp.float32)],
    )(policy_features, t_col, W_policy, bp_row, wt_row, bt_row, W_out, bo_11,
      edge_index)
